# baseline (speedup 1.0000x reference)
import functools

import jax
import jax.numpy as jnp
from jax import lax
from jax.experimental import pallas as pl
from jax.experimental.pallas import tpu as pltpu
from jax.experimental.pallas import tpu_sc as plsc

N = 1000000
D = 64
B = 16384

NC = 2
NS = 16
NW = NC * NS
B_PER_W = B // NW

_mesh = plsc.VectorSubcoreMesh(core_axis_name="c", subcore_axis_name="s")


@functools.partial(
    pl.kernel,
    mesh=_mesh,
    compiler_params=pltpu.CompilerParams(use_tc_tiling_on_sc=False),
    out_type=jax.ShapeDtypeStruct((D, B), jnp.float32),
    scratch_types=[
        pltpu.VMEM((B_PER_W,), jnp.int32),
        pltpu.VMEM((D, B_PER_W), jnp.float32),
        pltpu.SemaphoreType.DMA,
    ],
)
def _gather_kernel(tableT_hbm, idx_hbm, out_hbm, idx_v, cols_v, sem):
    wid = lax.axis_index("s") * NC + lax.axis_index("c")
    base = wid * B_PER_W
    pltpu.sync_copy(idx_hbm.at[pl.ds(base, B_PER_W)], idx_v)

    for c in range(D):
        pltpu.async_copy(tableT_hbm.at[c].at[idx_v], cols_v.at[c], sem)
    pltpu.make_async_copy(
        out_hbm.at[:, pl.ds(base, B_PER_W)], cols_v, sem
    ).wait()
    pltpu.sync_copy(cols_v, out_hbm.at[:, pl.ds(base, B_PER_W)])


def kernel(nodes, table):
    outT = _gather_kernel(table.T, nodes.astype(jnp.int32))
    return outT.T


# speedup vs baseline: 3.2244x; 3.2244x over previous
import functools

import jax
import jax.numpy as jnp
from jax import lax
from jax.experimental import pallas as pl
from jax.experimental.pallas import tpu as pltpu
from jax.experimental.pallas import tpu_sc as plsc

N = 1000000
D = 64
B = 16384

NC = 2
NS = 16
NW = NC * NS
B_PER_W = B // NW
R_PER_W = B_PER_W * D // 128

SLABS = N // 128
TAIL = N - SLABS * 128
SPW = (SLABS + NW - 1) // NW

_mesh = plsc.VectorSubcoreMesh(core_axis_name="c", subcore_axis_name="s")


@functools.partial(
    pl.kernel,
    mesh=_mesh,
    compiler_params=pltpu.CompilerParams(needs_layout_passes=False),
    out_type=jax.ShapeDtypeStruct((N // 2, 128), jnp.float32),
    scratch_types=[
        pltpu.VMEM((2, D, 128), jnp.float32),
        pltpu.VMEM((2, D, 128), jnp.float32),
        pltpu.SemaphoreType.DMA,
        pltpu.SemaphoreType.DMA,
    ],
)
def _transpose_kernel(tableT_hbm, tailp_hbm, out_hbm, in_v, tr_v, sem_in,
                      sem_out):
    wid = lax.axis_index("s") * NC + lax.axis_index("c")
    s0 = wid * SPW
    s1 = jnp.minimum(s0 + SPW, SLABS)

    lanes = lax.iota(jnp.int32, 16)

    def transpose_slab(src, dst, nq):
        def qbody(q, carry):
            for j in range(8):
                cvec = lanes + (16 * j) % 64
                hvec = jnp.full((16,), 2 * q + j // 4, jnp.int32)
                g = plsc.load_gather(src, [cvec, hvec])
                dst[q, pl.ds(16 * j, 16)] = g
            return carry

        lax.fori_loop(0, nq, qbody, 0)

    @pl.when(s0 < s1)
    def _():
        pltpu.async_copy(
            tableT_hbm.at[:, pl.ds(s0 * 128, 128)], in_v.at[s0 % 2], sem_in
        )

        def body(s, carry):
            buf = s % 2

            @pl.when(s + 1 < s1)
            def _():
                pltpu.async_copy(
                    tableT_hbm.at[:, pl.ds((s + 1) * 128, 128)],
                    in_v.at[(s + 1) % 2],
                    sem_in,
                )

            pltpu.make_async_copy(
                tableT_hbm.at[:, pl.ds(s * 128, 128)], in_v.at[buf], sem_in
            ).wait()

            @pl.when(s >= s0 + 2)
            def _():
                pltpu.make_async_copy(
                    tr_v.at[buf], out_hbm.at[pl.ds(s * 64, 64), :], sem_out
                ).wait()

            transpose_slab(in_v.at[buf], tr_v.at[buf], D)
            pltpu.async_copy(
                tr_v.at[buf], out_hbm.at[pl.ds(s * 64, 64), :], sem_out
            )
            return carry

        lax.fori_loop(s0, s1, body, 0)

        ntail = jnp.minimum(s1 - s0, 2) * (64 * 128 * 4)

        @pl.when(ntail > 0)
        def _():
            pltpu.make_async_copy(
                tr_v.at[0], out_hbm.at[pl.ds(0, 64), :], sem_out
            ).wait()

        @pl.when(ntail > 64 * 128 * 4)
        def _():
            pltpu.make_async_copy(
                tr_v.at[0], out_hbm.at[pl.ds(0, 64), :], sem_out
            ).wait()

    @pl.when(wid == 0)
    def _():
        pltpu.sync_copy(tailp_hbm, in_v.at[0])
        transpose_slab(in_v.at[0], tr_v.at[0], TAIL // 2)
        pltpu.sync_copy(
            tr_v.at[0, pl.ds(0, TAIL // 2)],
            out_hbm.at[pl.ds(SLABS * 64, TAIL // 2), :],
        )


@functools.partial(
    pl.kernel,
    mesh=_mesh,
    compiler_params=pltpu.CompilerParams(needs_layout_passes=False),
    out_type=jax.ShapeDtypeStruct((B * D // 128, 128), jnp.float32),
    scratch_types=[
        pltpu.VMEM((B_PER_W,), jnp.int32),
        pltpu.VMEM((B_PER_W,), jnp.int32),
        pltpu.VMEM((B_PER_W,), jnp.int32),
        pltpu.VMEM((B_PER_W, 128), jnp.float32),
        pltpu.VMEM((R_PER_W, 128), jnp.float32),
        pltpu.SemaphoreType.DMA,
    ],
)
def _gather_kernel(packed_hbm, idx_hbm, out_hbm, idx_v, j_v, par_v, rows_v,
                   blk_v, sem):
    wid = lax.axis_index("s") * NC + lax.axis_index("c")
    base = wid * B_PER_W
    pltpu.sync_copy(idx_hbm.at[pl.ds(base, B_PER_W)], idx_v)

    def mk_idx(g, carry):
        v = idx_v[pl.ds(g * 16, 16)]
        j_v[pl.ds(g * 16, 16)] = v >> 1
        par_v[pl.ds(g * 16, 16)] = (v & 1) * 64
        return carry

    lax.fori_loop(0, B_PER_W // 16, mk_idx, 0)
    pltpu.async_copy(packed_hbm.at[j_v], rows_v, sem).wait()

    lanes = lax.iota(jnp.int32, 16)

    def extract(g, carry):
        kvec = lanes + g * 16
        par = par_v[pl.ds(g * 16, 16)]
        rvec = kvec >> 1
        cbase = (kvec & 1) * 64
        for c in range(D):
            val = plsc.load_gather(rows_v, [kvec, par + c])
            plsc.store_scatter(blk_v, [rvec, cbase + c], val)
        return carry

    lax.fori_loop(0, B_PER_W // 16, extract, 0)
    pltpu.sync_copy(blk_v, out_hbm.at[pl.ds(wid * R_PER_W, R_PER_W)])


def kernel(nodes, table):
    tableT = table.T
    tail_pad = jnp.pad(tableT[:, SLABS * 128:], ((0, 0), (0, 128 - TAIL)))
    packed = _transpose_kernel(tableT, tail_pad)
    out = _gather_kernel(packed, nodes.astype(jnp.int32))
    return out.reshape(B, D)


# speedup vs baseline: 5.8381x; 1.8106x over previous
import functools

import jax
import jax.numpy as jnp
from jax import lax
from jax.experimental import pallas as pl
from jax.experimental.pallas import tpu as pltpu
from jax.experimental.pallas import tpu_sc as plsc

N = 1000000
D = 64
B = 16384

NC = 2
NS = 16
NW = NC * NS
B_PER_W = B // NW
R_PER_W = B_PER_W * D // 128

SLABS = N // 128
TAIL = N - SLABS * 128
SPW = (SLABS + NW - 1) // NW

_mesh = plsc.VectorSubcoreMesh(core_axis_name="c", subcore_axis_name="s")


@functools.partial(
    pl.kernel,
    mesh=_mesh,
    compiler_params=pltpu.CompilerParams(needs_layout_passes=False),
    out_type=jax.ShapeDtypeStruct((N // 2, 128), jnp.float32),
    scratch_types=[
        pltpu.VMEM((2, D, 128), jnp.float32),
        pltpu.VMEM((2, D, 128), jnp.float32),
        pltpu.SemaphoreType.DMA,
        pltpu.SemaphoreType.DMA,
    ],
)
def _transpose_kernel(tableT_hbm, tailp_hbm, out_hbm, in_v, tr_v, sem_in,
                      sem_out):
    wid = lax.axis_index("s") * NC + lax.axis_index("c")
    s0 = wid * SPW
    s1 = jnp.minimum(s0 + SPW, SLABS)

    lanes = lax.iota(jnp.int32, 16)
    cvecs = [lanes + 16 * jj for jj in range(4)]

    def transpose_slab(src, dst, nq):
        @plsc.parallel_loop(0, nq, 1, unroll=4)
        def qbody(q):
            h0 = jnp.full((16,), 2 * q, jnp.int32)
            h1 = h0 + 1
            for j in range(8):
                g = plsc.load_gather(src, [cvecs[j % 4], h0 if j < 4 else h1])
                dst[q, pl.ds(16 * j, 16)] = g

    @pl.when(s0 < s1)
    def _():
        pltpu.async_copy(
            tableT_hbm.at[:, pl.ds(s0 * 128, 128)], in_v.at[s0 % 2], sem_in
        )

        def body(s, carry):
            buf = s % 2

            @pl.when(s + 1 < s1)
            def _():
                pltpu.async_copy(
                    tableT_hbm.at[:, pl.ds((s + 1) * 128, 128)],
                    in_v.at[(s + 1) % 2],
                    sem_in,
                )

            pltpu.make_async_copy(
                tableT_hbm.at[:, pl.ds(s * 128, 128)], in_v.at[buf], sem_in
            ).wait()

            @pl.when(s >= s0 + 2)
            def _():
                pltpu.make_async_copy(
                    tr_v.at[buf], out_hbm.at[pl.ds(s * 64, 64), :], sem_out
                ).wait()

            transpose_slab(in_v.at[buf], tr_v.at[buf], D)
            pltpu.async_copy(
                tr_v.at[buf], out_hbm.at[pl.ds(s * 64, 64), :], sem_out
            )
            return carry

        lax.fori_loop(s0, s1, body, 0)

        ntail = jnp.minimum(s1 - s0, 2) * (64 * 128 * 4)

        @pl.when(ntail > 0)
        def _():
            pltpu.make_async_copy(
                tr_v.at[0], out_hbm.at[pl.ds(0, 64), :], sem_out
            ).wait()

        @pl.when(ntail > 64 * 128 * 4)
        def _():
            pltpu.make_async_copy(
                tr_v.at[0], out_hbm.at[pl.ds(0, 64), :], sem_out
            ).wait()

    @pl.when(wid == 0)
    def _():
        pltpu.sync_copy(tailp_hbm, in_v.at[0])
        transpose_slab(in_v.at[0], tr_v.at[0], TAIL // 2)
        pltpu.sync_copy(
            tr_v.at[0, pl.ds(0, TAIL // 2)],
            out_hbm.at[pl.ds(SLABS * 64, TAIL // 2), :],
        )


@functools.partial(
    pl.kernel,
    mesh=_mesh,
    compiler_params=pltpu.CompilerParams(needs_layout_passes=False),
    out_type=jax.ShapeDtypeStruct((B * D // 128, 128), jnp.float32),
    scratch_types=[
        pltpu.VMEM((B_PER_W,), jnp.int32),
        pltpu.VMEM((B_PER_W,), jnp.int32),
        pltpu.VMEM((B_PER_W,), jnp.int32),
        pltpu.VMEM((B_PER_W, 128), jnp.float32),
        pltpu.VMEM((R_PER_W, 128), jnp.float32),
        pltpu.SemaphoreType.DMA,
    ],
)
def _gather_kernel(packed_hbm, idx_hbm, out_hbm, idx_v, j_v, par_v, rows_v,
                   blk_v, sem):
    wid = lax.axis_index("s") * NC + lax.axis_index("c")
    base = wid * B_PER_W
    pltpu.sync_copy(idx_hbm.at[pl.ds(base, B_PER_W)], idx_v)

    def mk_idx(g, carry):
        v = idx_v[pl.ds(g * 16, 16)]
        j_v[pl.ds(g * 16, 16)] = v >> 1
        par_v[pl.ds(g * 16, 16)] = (v & 1) * 64
        return carry

    lax.fori_loop(0, B_PER_W // 16, mk_idx, 0)
    pltpu.async_copy(packed_hbm.at[j_v], rows_v, sem).wait()

    lanes = lax.iota(jnp.int32, 16)

    def extract(g, carry):
        kvec = lanes + g * 16
        par = par_v[pl.ds(g * 16, 16)]
        rvec = kvec >> 1
        cbase = (kvec & 1) * 64
        for c in range(D):
            val = plsc.load_gather(rows_v, [kvec, par + c])
            plsc.store_scatter(blk_v, [rvec, cbase + c], val)
        return carry

    lax.fori_loop(0, B_PER_W // 16, extract, 0)
    pltpu.sync_copy(blk_v, out_hbm.at[pl.ds(wid * R_PER_W, R_PER_W)])


def kernel(nodes, table):
    tableT = table.T
    tail_pad = jnp.pad(tableT[:, SLABS * 128:], ((0, 0), (0, 128 - TAIL)))
    packed = _transpose_kernel(tableT, tail_pad)
    out = _gather_kernel(packed, nodes.astype(jnp.int32))
    return out.reshape(B, D)


# speedup vs baseline: 5.8387x; 1.0001x over previous
import functools

import jax
import jax.numpy as jnp
from jax import lax
from jax.experimental import pallas as pl
from jax.experimental.pallas import tpu as pltpu
from jax.experimental.pallas import tpu_sc as plsc

N = 1000000
D = 64
B = 16384

NC = 2
NS = 16
NW = NC * NS
B_PER_W = B // NW
R_PER_W = B_PER_W * D // 128

SLABS = N // 128
TAIL = N - SLABS * 128
SPW = (SLABS + NW - 1) // NW

_mesh = plsc.VectorSubcoreMesh(core_axis_name="c", subcore_axis_name="s")


@functools.partial(
    pl.kernel,
    mesh=_mesh,
    compiler_params=pltpu.CompilerParams(needs_layout_passes=False),
    out_type=jax.ShapeDtypeStruct((N // 2, 128), jnp.float32),
    scratch_types=[
        pltpu.VMEM((2, D, 128), jnp.float32),
        pltpu.VMEM((2, D, 128), jnp.float32),
        pltpu.SemaphoreType.DMA,
        pltpu.SemaphoreType.DMA,
    ],
)
def _transpose_kernel(tableT_hbm, tailp_hbm, out_hbm, in_v, tr_v, sem_in,
                      sem_out):
    wid = lax.axis_index("s") * NC + lax.axis_index("c")
    s0 = wid * SPW
    s1 = jnp.minimum(s0 + SPW, SLABS)

    lanes = lax.iota(jnp.int32, 16)
    cvecs = [lanes + 16 * jj for jj in range(4)]

    def transpose_slab(src, dst, nq):
        @plsc.parallel_loop(0, nq, 1, unroll=8)
        def qbody(q):
            h0 = jnp.full((16,), 2 * q, jnp.int32)
            h1 = h0 + 1
            for j in range(8):
                g = plsc.load_gather(src, [cvecs[j % 4], h0 if j < 4 else h1])
                dst[q, pl.ds(16 * j, 16)] = g

    @pl.when(s0 < s1)
    def _():
        pltpu.async_copy(
            tableT_hbm.at[:, pl.ds(s0 * 128, 128)], in_v.at[s0 % 2], sem_in
        )

        def body(s, carry):
            buf = s % 2

            @pl.when(s + 1 < s1)
            def _():
                pltpu.async_copy(
                    tableT_hbm.at[:, pl.ds((s + 1) * 128, 128)],
                    in_v.at[(s + 1) % 2],
                    sem_in,
                )

            pltpu.make_async_copy(
                tableT_hbm.at[:, pl.ds(s * 128, 128)], in_v.at[buf], sem_in
            ).wait()

            @pl.when(s >= s0 + 2)
            def _():
                pltpu.make_async_copy(
                    tr_v.at[buf], out_hbm.at[pl.ds(s * 64, 64), :], sem_out
                ).wait()

            transpose_slab(in_v.at[buf], tr_v.at[buf], D)
            pltpu.async_copy(
                tr_v.at[buf], out_hbm.at[pl.ds(s * 64, 64), :], sem_out
            )
            return carry

        lax.fori_loop(s0, s1, body, 0)

        ntail = jnp.minimum(s1 - s0, 2) * (64 * 128 * 4)

        @pl.when(ntail > 0)
        def _():
            pltpu.make_async_copy(
                tr_v.at[0], out_hbm.at[pl.ds(0, 64), :], sem_out
            ).wait()

        @pl.when(ntail > 64 * 128 * 4)
        def _():
            pltpu.make_async_copy(
                tr_v.at[0], out_hbm.at[pl.ds(0, 64), :], sem_out
            ).wait()

    @pl.when(wid == 0)
    def _():
        pltpu.sync_copy(tailp_hbm, in_v.at[0])
        transpose_slab(in_v.at[0], tr_v.at[0], TAIL // 2)
        pltpu.sync_copy(
            tr_v.at[0, pl.ds(0, TAIL // 2)],
            out_hbm.at[pl.ds(SLABS * 64, TAIL // 2), :],
        )


@functools.partial(
    pl.kernel,
    mesh=_mesh,
    compiler_params=pltpu.CompilerParams(needs_layout_passes=False),
    out_type=jax.ShapeDtypeStruct((B * D // 128, 128), jnp.float32),
    scratch_types=[
        pltpu.VMEM((B_PER_W,), jnp.int32),
        pltpu.VMEM((B_PER_W,), jnp.int32),
        pltpu.VMEM((B_PER_W,), jnp.int32),
        pltpu.VMEM((B_PER_W, 128), jnp.float32),
        pltpu.VMEM((R_PER_W, 128), jnp.float32),
        pltpu.SemaphoreType.DMA,
    ],
)
def _gather_kernel(packed_hbm, idx_hbm, out_hbm, idx_v, j_v, par_v, rows_v,
                   blk_v, sem):
    wid = lax.axis_index("s") * NC + lax.axis_index("c")
    base = wid * B_PER_W
    pltpu.sync_copy(idx_hbm.at[pl.ds(base, B_PER_W)], idx_v)

    def mk_idx(g, carry):
        v = idx_v[pl.ds(g * 16, 16)]
        j_v[pl.ds(g * 16, 16)] = v >> 1
        par_v[pl.ds(g * 16, 16)] = (v & 1) * 64
        return carry

    lax.fori_loop(0, B_PER_W // 16, mk_idx, 0)
    pltpu.async_copy(packed_hbm.at[j_v], rows_v, sem).wait()

    lanes = lax.iota(jnp.int32, 16)

    def extract(g, carry):
        kvec = lanes + g * 16
        par = par_v[pl.ds(g * 16, 16)]
        rvec = kvec >> 1
        cbase = (kvec & 1) * 64
        for c in range(D):
            val = plsc.load_gather(rows_v, [kvec, par + c])
            plsc.store_scatter(blk_v, [rvec, cbase + c], val)
        return carry

    lax.fori_loop(0, B_PER_W // 16, extract, 0)
    pltpu.sync_copy(blk_v, out_hbm.at[pl.ds(wid * R_PER_W, R_PER_W)])


def kernel(nodes, table):
    tableT = table.T
    tail_pad = jnp.pad(tableT[:, SLABS * 128:], ((0, 0), (0, 128 - TAIL)))
    packed = _transpose_kernel(tableT, tail_pad)
    out = _gather_kernel(packed, nodes.astype(jnp.int32))
    return out.reshape(B, D)


# speedup vs baseline: 7.5523x; 1.2935x over previous
import functools

import jax
import jax.numpy as jnp
from jax import lax
from jax.experimental import pallas as pl
from jax.experimental.pallas import tpu as pltpu
from jax.experimental.pallas import tpu_sc as plsc

N = 1000000
D = 64
B = 16384

NC = 2
NS = 16
NW = NC * NS
B_PER_W = B // NW
R_PER_W = B_PER_W * D // 128

SLABS = N // 128
TAIL = N - SLABS * 128
SPW = (SLABS + NW - 1) // NW

_mesh = plsc.VectorSubcoreMesh(core_axis_name="c", subcore_axis_name="s")


@functools.partial(
    pl.kernel,
    mesh=_mesh,
    compiler_params=pltpu.CompilerParams(needs_layout_passes=False),
    out_type=jax.ShapeDtypeStruct((N // 2, 128), jnp.float32),
    scratch_types=[
        pltpu.VMEM((2, D, 128), jnp.float32),
        pltpu.VMEM((2, D, 128), jnp.float32),
        pltpu.SemaphoreType.DMA,
        pltpu.SemaphoreType.DMA,
    ],
)
def _transpose_kernel(tableT_hbm, tailp_hbm, out_hbm, in_v, tr_v, sem_in,
                      sem_out):
    wid = lax.axis_index("s") * NC + lax.axis_index("c")
    s0 = wid * SPW
    s1 = jnp.minimum(s0 + SPW, SLABS)

    lanes = lax.iota(jnp.int32, 16)
    cvecs = [lanes + 16 * jj for jj in range(4)]

    def transpose_slab(src, dst, nq):
        @plsc.parallel_loop(0, nq, 1, unroll=8)
        def qbody(q):
            h0 = jnp.full((16,), 2 * q, jnp.int32)
            h1 = h0 + 1
            for j in range(8):
                g = plsc.load_gather(src, [cvecs[j % 4], h0 if j < 4 else h1])
                dst[q, pl.ds(16 * j, 16)] = g

    @pl.when(s0 < s1)
    def _():
        pltpu.async_copy(
            tableT_hbm.at[:, pl.ds(s0 * 128, 128)], in_v.at[s0 % 2], sem_in
        )

        def body(s, carry):
            buf = s % 2

            @pl.when(s + 1 < s1)
            def _():
                pltpu.async_copy(
                    tableT_hbm.at[:, pl.ds((s + 1) * 128, 128)],
                    in_v.at[(s + 1) % 2],
                    sem_in,
                )

            pltpu.make_async_copy(
                tableT_hbm.at[:, pl.ds(s * 128, 128)], in_v.at[buf], sem_in
            ).wait()

            @pl.when(s >= s0 + 2)
            def _():
                pltpu.make_async_copy(
                    tr_v.at[buf], out_hbm.at[pl.ds(s * 64, 64), :], sem_out
                ).wait()

            transpose_slab(in_v.at[buf], tr_v.at[buf], D)
            pltpu.async_copy(
                tr_v.at[buf], out_hbm.at[pl.ds(s * 64, 64), :], sem_out
            )
            return carry

        lax.fori_loop(s0, s1, body, 0)

        ntail = jnp.minimum(s1 - s0, 2) * (64 * 128 * 4)

        @pl.when(ntail > 0)
        def _():
            pltpu.make_async_copy(
                tr_v.at[0], out_hbm.at[pl.ds(0, 64), :], sem_out
            ).wait()

        @pl.when(ntail > 64 * 128 * 4)
        def _():
            pltpu.make_async_copy(
                tr_v.at[0], out_hbm.at[pl.ds(0, 64), :], sem_out
            ).wait()

    @pl.when(wid == 0)
    def _():
        pltpu.sync_copy(tailp_hbm, in_v.at[0])
        transpose_slab(in_v.at[0], tr_v.at[0], TAIL // 2)
        pltpu.sync_copy(
            tr_v.at[0, pl.ds(0, TAIL // 2)],
            out_hbm.at[pl.ds(SLABS * 64, TAIL // 2), :],
        )


@functools.partial(
    pl.kernel,
    mesh=_mesh,
    compiler_params=pltpu.CompilerParams(needs_layout_passes=False),
    out_type=jax.ShapeDtypeStruct((B * D // 128, 128), jnp.float32),
    scratch_types=[
        pltpu.VMEM((B_PER_W,), jnp.int32),
        pltpu.VMEM((B_PER_W,), jnp.int32),
        pltpu.VMEM((B_PER_W,), jnp.int32),
        pltpu.VMEM((B_PER_W, 128), jnp.float32),
        pltpu.VMEM((R_PER_W, 128), jnp.float32),
        pltpu.SemaphoreType.DMA,
    ],
)
def _gather_kernel(packed_hbm, idx_hbm, out_hbm, idx_v, j_v, par_v, rows_v,
                   blk_v, sem):
    wid = lax.axis_index("s") * NC + lax.axis_index("c")
    base = wid * B_PER_W
    pltpu.sync_copy(idx_hbm.at[pl.ds(base, B_PER_W)], idx_v)

    def mk_idx(g, carry):
        v = idx_v[pl.ds(g * 16, 16)]
        j_v[pl.ds(g * 16, 16)] = v >> 1
        par_v[pl.ds(g * 16, 16)] = (v & 1) * 64
        return carry

    lax.fori_loop(0, B_PER_W // 16, mk_idx, 0)
    pltpu.async_copy(packed_hbm.at[j_v], rows_v, sem).wait()

    lanes = lax.iota(jnp.int32, 16)

    def extract(g, carry):
        kvec = lanes + g * 16
        par = par_v[pl.ds(g * 16, 16)]
        rvec = kvec >> 1
        cbase = (kvec & 1) * 64
        for c in range(D):
            val = plsc.load_gather(rows_v, [kvec, par + c])
            plsc.store_scatter(blk_v, [rvec, cbase + c], val)
        return carry

    lax.fori_loop(0, B_PER_W // 16, extract, 0)
    pltpu.sync_copy(blk_v, out_hbm.at[pl.ds(wid * R_PER_W, R_PER_W)])


def kernel(nodes, table):
    packed = table.reshape(N // 2, 128)
    out = _gather_kernel(packed, nodes.astype(jnp.int32))
    return out.reshape(B, D)
